# baseline (speedup 1.0000x reference)
import functools

import jax
import jax.numpy as jnp
from jax import lax
from jax.experimental import pallas as pl
from jax.experimental.pallas import tpu as pltpu
from jax.experimental.pallas import tpu_sc as plsc

B0, B1, D = 16384, 200, 22
TOTAL = B0 * B1 * D
NC, NS = 2, 16
NW = NC * NS
PER_W = TOTAL // NW
LANES = 16
PERIOD = 176
NVREG = PERIOD // LANES
CHUNK = 112_640
REPS = CHUNK // PERIOD
NCHUNK = PER_W // CHUNK

assert PER_W % CHUNK == 0 and CHUNK % PERIOD == 0 and PERIOD % D == 0
assert PER_W % 8 == 0 and CHUNK % 8 == 0


@functools.partial(
    pl.kernel,
    out_type=jax.ShapeDtypeStruct((TOTAL,), jnp.float32),
    mesh=plsc.VectorSubcoreMesh(core_axis_name="c", subcore_axis_name="s"),
    scratch_types=[
        pltpu.VMEM((32,), jnp.float32),
        pltpu.VMEM((CHUNK,), jnp.float32),
        pltpu.SemaphoreType.DMA,
    ],
)
def _emb_broadcast(table_hbm, out_hbm, tbl_v, buf_v, sem):
    wid = lax.axis_index("s") * NC + lax.axis_index("c")
    base = wid * PER_W

    pltpu.sync_copy(table_hbm, tbl_v)

    lane = lax.iota(jnp.int32, LANES)
    a = tbl_v[pl.ds(0, LANES)]
    b = tbl_v[pl.ds(LANES, LANES)]

    def take16(vec, idx):
        dnums = lax.GatherDimensionNumbers(
            offset_dims=(), collapsed_slice_dims=(0,), start_index_map=(0,)
        )
        return lax.gather(
            vec,
            idx[:, None],
            dnums,
            slice_sizes=(1,),
            mode=lax.GatherScatterMode.PROMISE_IN_BOUNDS,
        )

    vregs = []
    for i in range(NVREG):
        idx = lax.rem(lane + (i * LANES) % D, D)
        va = take16(a, jnp.minimum(idx, LANES - 1))
        vb = take16(b, jnp.maximum(idx - LANES, 0))
        vregs.append(jnp.where(idx < LANES, va, vb))

    def fill(j, carry):
        off = j * PERIOD
        for i in range(NVREG):
            buf_v[pl.ds(off + i * LANES, LANES)] = vregs[i]
        return carry

    lax.fori_loop(0, REPS, fill, 0, unroll=2)

    descs = [
        pltpu.async_copy(
            buf_v, out_hbm.at[pl.ds(base + c * CHUNK, CHUNK)], sem
        )
        for c in range(NCHUNK)
    ]
    for d in descs:
        d.wait()


def kernel(input, table):
    del input
    tbl32 = jnp.pad(table.reshape(-1), (0, 32 - D))
    out = _emb_broadcast(tbl32)
    return out.reshape(B0, B1, D)


# speedup vs baseline: 24.2883x; 24.2883x over previous
import functools

import jax
import jax.numpy as jnp
from jax import lax
from jax.experimental import pallas as pl
from jax.experimental.pallas import tpu as pltpu
from jax.experimental.pallas import tpu_sc as plsc

B0, B1, D = 16384, 200, 22
N = B0 * B1
TOTAL = N * D
NC, NS = 2, 16
NW = NC * NS
LANES = 16
CH = 51_200
RUN_CH = N // CH
W_CH = TOTAL // (NW * CH)

assert RUN_CH * CH == N and NW * W_CH * CH == TOTAL


@functools.partial(
    pl.kernel,
    out_type=jax.ShapeDtypeStruct((TOTAL,), jnp.float32),
    mesh=plsc.VectorSubcoreMesh(core_axis_name="c", subcore_axis_name="s"),
    scratch_types=[
        pltpu.VMEM((32,), jnp.float32),
        pltpu.VMEM((CH,), jnp.float32),
        pltpu.VMEM((CH,), jnp.float32),
        pltpu.SemaphoreType.DMA,
    ],
)
def _emb_broadcast(table_hbm, out_hbm, tbl_v, buf_a, buf_b, sem):
    wid = lax.axis_index("s") * NC + lax.axis_index("c")
    base_g = wid * W_CH
    k0 = base_g // RUN_CH
    k1 = (base_g + W_CH - 1) // RUN_CH
    n_a = jnp.minimum((k0 + 1) * RUN_CH, base_g + W_CH) - base_g

    pltpu.sync_copy(table_hbm, tbl_v)
    a = tbl_v[pl.ds(0, LANES)]
    b = tbl_v[pl.ds(LANES, LANES)]

    def take16(vec, idx):
        dnums = lax.GatherDimensionNumbers(
            offset_dims=(), collapsed_slice_dims=(0,), start_index_map=(0,)
        )
        return lax.gather(
            vec,
            idx[:, None],
            dnums,
            slice_sizes=(1,),
            mode=lax.GatherScatterMode.PROMISE_IN_BOUNDS,
        )

    def splat(k):
        bk = jnp.zeros((LANES,), jnp.int32) + k
        va = take16(a, jnp.minimum(bk, LANES - 1))
        vb = take16(b, jnp.maximum(bk - LANES, 0))
        return jnp.where(bk < LANES, va, vb)

    def fill(buf, vec):
        def body(c, carry):
            buf[pl.ds(c * LANES, LANES)] = vec
            return carry

        lax.fori_loop(0, CH // LANES, body, 0, unroll=8)

    def fire(buf, lo, hi):
        def body(c, carry):
            pltpu.async_copy(buf, out_hbm.at[pl.ds(c * CH, CH)], sem)
            return carry

        lax.fori_loop(lo, hi, body, 0)

    fill(buf_a, splat(k0))
    fire(buf_a, base_g, base_g + n_a)
    fill(buf_b, splat(k1))
    fire(buf_b, base_g + n_a, base_g + W_CH)

    def drain(c, carry):
        pltpu.make_async_copy(buf_a, out_hbm.at[pl.ds(0, CH)], sem).wait()
        return carry

    lax.fori_loop(0, W_CH, drain, 0)


def kernel(input, table):
    del input
    tbl32 = jnp.pad(table.reshape(-1), (0, 32 - D))
    flat = _emb_broadcast(tbl32)
    out5 = flat.reshape(D, B1 // 8, B0 // 128, 8, 128)
    return out5.transpose(2, 4, 1, 3, 0).reshape(B0, B1, D)


# speedup vs baseline: 24.5149x; 1.0093x over previous
import functools

import jax
import jax.numpy as jnp
from jax import lax
from jax.experimental import pallas as pl
from jax.experimental.pallas import tpu as pltpu
from jax.experimental.pallas import tpu_sc as plsc

B0, B1, D = 16384, 200, 22
N = B0 * B1
TOTAL = N * D
NC, NS = 2, 16
NW = NC * NS
LANES = 16
CH = 51_200
RUN_CH = N // CH
W_CH = TOTAL // (NW * CH)

assert RUN_CH * CH == N and NW * W_CH * CH == TOTAL


@functools.partial(
    pl.kernel,
    out_type=jax.ShapeDtypeStruct((TOTAL,), jnp.float32),
    mesh=plsc.VectorSubcoreMesh(core_axis_name="c", subcore_axis_name="s"),
    scratch_types=[
        pltpu.VMEM((32,), jnp.float32),
        pltpu.VMEM((CH,), jnp.float32),
        pltpu.VMEM((CH,), jnp.float32),
        pltpu.SemaphoreType.DMA,
    ],
)
def _emb_broadcast(table_hbm, out_hbm, tbl_v, buf_a, buf_b, sem):
    wid = lax.axis_index("s") * NC + lax.axis_index("c")
    base_g = wid * W_CH
    k0 = base_g // RUN_CH
    k1 = (base_g + W_CH - 1) // RUN_CH
    n_a = jnp.minimum((k0 + 1) * RUN_CH, base_g + W_CH) - base_g

    pltpu.sync_copy(table_hbm, tbl_v.at[pl.ds(0, D)])
    a = tbl_v[pl.ds(0, LANES)]
    b = tbl_v[pl.ds(LANES, LANES)]

    def take16(vec, idx):
        dnums = lax.GatherDimensionNumbers(
            offset_dims=(), collapsed_slice_dims=(0,), start_index_map=(0,)
        )
        return lax.gather(
            vec,
            idx[:, None],
            dnums,
            slice_sizes=(1,),
            mode=lax.GatherScatterMode.PROMISE_IN_BOUNDS,
        )

    def splat(k):
        bk = jnp.zeros((LANES,), jnp.int32) + k
        va = take16(a, jnp.minimum(bk, LANES - 1))
        vb = take16(b, jnp.maximum(bk - LANES, 0))
        return jnp.where(bk < LANES, va, vb)

    def fill(buf, vec):
        def body(c, carry):
            buf[pl.ds(c * LANES, LANES)] = vec
            return carry

        lax.fori_loop(0, CH // LANES, body, 0, unroll=8)

    def fire(buf, lo, hi):
        def body(c, carry):
            pltpu.async_copy(buf, out_hbm.at[pl.ds(c * CH, CH)], sem)
            return carry

        lax.fori_loop(lo, hi, body, 0)

    PIECE = CH // 4
    va0 = splat(k0)
    for p in range(4):
        def piece_body(c, carry):
            buf_a[pl.ds(c * LANES, LANES)] = va0
            return carry

        lax.fori_loop(p * (PIECE // LANES), (p + 1) * (PIECE // LANES),
                      piece_body, 0, unroll=8)
        pltpu.async_copy(
            buf_a.at[pl.ds(p * PIECE, PIECE)],
            out_hbm.at[pl.ds(base_g * CH + p * PIECE, PIECE)],
            sem,
        )
    fire(buf_a, base_g + 1, base_g + n_a)
    fill(buf_b, splat(k1))
    fire(buf_b, base_g + n_a, base_g + W_CH)

    def drain_piece(c, carry):
        pltpu.make_async_copy(
            buf_a.at[pl.ds(0, PIECE)], out_hbm.at[pl.ds(0, PIECE)], sem
        ).wait()
        return carry

    lax.fori_loop(0, 4, drain_piece, 0)

    def drain(c, carry):
        pltpu.make_async_copy(buf_a, out_hbm.at[pl.ds(0, CH)], sem).wait()
        return carry

    lax.fori_loop(0, W_CH - 1, drain, 0)


def kernel(input, table):
    del input
    flat = _emb_broadcast(table.reshape(-1))
    out5 = flat.reshape(D, B1 // 8, B0 // 128, 8, 128)
    return out5.transpose(2, 4, 1, 3, 0).reshape(B0, B1, D)


# speedup vs baseline: 24.5829x; 1.0028x over previous
import functools

import jax
import jax.numpy as jnp
from jax import lax
from jax.experimental import pallas as pl
from jax.experimental.pallas import tpu as pltpu
from jax.experimental.pallas import tpu_sc as plsc

B0, B1, D = 16384, 200, 22
N = B0 * B1
TOTAL = N * D
NC, NS = 2, 16
NW = NC * NS
LANES = 16
CH = 51_200
RUN_CH = N // CH
PAIR_CH = TOTAL // (NS * CH)
C0_CH = 43

assert RUN_CH * CH == N and NS * PAIR_CH * CH == TOTAL
assert C0_CH < RUN_CH and PAIR_CH - C0_CH < RUN_CH


@functools.partial(
    pl.kernel,
    out_type=jax.ShapeDtypeStruct((TOTAL,), jnp.float32),
    mesh=plsc.VectorSubcoreMesh(core_axis_name="c", subcore_axis_name="s"),
    scratch_types=[
        pltpu.VMEM((32,), jnp.float32),
        pltpu.VMEM((CH,), jnp.float32),
        pltpu.VMEM((CH,), jnp.float32),
        pltpu.SemaphoreType.DMA,
    ],
)
def _emb_broadcast(table_hbm, out_hbm, tbl_v, buf_a, buf_b, sem):
    cid = lax.axis_index("c")
    base_g = lax.axis_index("s") * PAIR_CH + cid * C0_CH
    n_g = jnp.where(cid == 0, C0_CH, PAIR_CH - C0_CH)
    k0 = base_g // RUN_CH
    k1 = (base_g + n_g - 1) // RUN_CH
    n_a = jnp.minimum((k0 + 1) * RUN_CH, base_g + n_g) - base_g

    pltpu.sync_copy(table_hbm, tbl_v.at[pl.ds(0, D)])
    a = tbl_v[pl.ds(0, LANES)]
    b = tbl_v[pl.ds(LANES, LANES)]

    def take16(vec, idx):
        dnums = lax.GatherDimensionNumbers(
            offset_dims=(), collapsed_slice_dims=(0,), start_index_map=(0,)
        )
        return lax.gather(
            vec,
            idx[:, None],
            dnums,
            slice_sizes=(1,),
            mode=lax.GatherScatterMode.PROMISE_IN_BOUNDS,
        )

    def splat(k):
        bk = jnp.zeros((LANES,), jnp.int32) + k
        va = take16(a, jnp.minimum(bk, LANES - 1))
        vb = take16(b, jnp.maximum(bk - LANES, 0))
        return jnp.where(bk < LANES, va, vb)

    def fill(buf, vec):
        def body(c, carry):
            buf[pl.ds(c * LANES, LANES)] = vec
            return carry

        lax.fori_loop(0, CH // LANES, body, 0, unroll=8)

    def fire(buf, lo, hi):
        def body(c, carry):
            pltpu.async_copy(buf, out_hbm.at[pl.ds(c * CH, CH)], sem)
            return carry

        lax.fori_loop(lo, hi, body, 0)

    PIECE = CH // 4
    va0 = splat(k0)
    for p in range(4):
        def piece_body(c, carry):
            buf_a[pl.ds(c * LANES, LANES)] = va0
            return carry

        lax.fori_loop(p * (PIECE // LANES), (p + 1) * (PIECE // LANES),
                      piece_body, 0, unroll=8)
        pltpu.async_copy(
            buf_a.at[pl.ds(p * PIECE, PIECE)],
            out_hbm.at[pl.ds(base_g * CH + p * PIECE, PIECE)],
            sem,
        )
    fire(buf_a, base_g + 1, base_g + n_a)
    fill(buf_b, splat(k1))
    fire(buf_b, base_g + n_a, base_g + n_g)

    def drain_piece(c, carry):
        pltpu.make_async_copy(
            buf_a.at[pl.ds(0, PIECE)], out_hbm.at[pl.ds(0, PIECE)], sem
        ).wait()
        return carry

    lax.fori_loop(0, 4, drain_piece, 0)

    def drain(c, carry):
        pltpu.make_async_copy(buf_a, out_hbm.at[pl.ds(0, CH)], sem).wait()
        return carry

    lax.fori_loop(0, n_g - 1, drain, 0)


def kernel(input, table):
    del input
    flat = _emb_broadcast(table.reshape(-1))
    out5 = flat.reshape(D, B1 // 8, B0 // 128, 8, 128)
    return out5.transpose(2, 4, 1, 3, 0).reshape(B0, B1, D)
